# baseline (speedup 1.0000x reference)
import functools

import jax
import jax.numpy as jnp
from jax import lax
from jax.experimental import pallas as pl
from jax.experimental.pallas import tpu as pltpu, tpu_sc as plsc

N_NODES = 10000
D = 128
HALF = 64
W = HALF + 16
E = 320000
C = 100
K = 20
SUPER = K * C
N_SUB = 16
SUPER_PER_TILE = E // SUPER // N_SUB
ROWS_PER_TILE = N_NODES // N_SUB
EP = 25


def _sc_body(feat_hbm, src_hbm, dst_hbm, w_hbm, out_hbm,
             srcbuf, dstbuf, wbuf, gbuf0, gbuf1, rowbuf0, rowbuf1,
             selacc, selfeat, sem0, sem1, ssem0, ssem1, acc_sh, feat_sh):
    c = lax.axis_index("c")
    s = lax.axis_index("s")
    r0 = ROWS_PER_TILE * s
    feat_base = c * N_NODES

    zeros16 = jnp.zeros((16,), jnp.float32)
    iota16 = lax.iota(jnp.int32, 16)
    e0 = jnp.where(iota16 == 0, 1.0, 0.0).astype(jnp.float32)
    gbufs = (gbuf0, gbuf1)
    sems = (sem0, sem1)
    rowbufs = (rowbuf0, rowbuf1)
    ssems = (ssem0, ssem1)

    def init_row(i, carry):
        rowbuf0[i, pl.ds(HALF, 16)] = e0
        rowbuf1[i, pl.ds(HALF, 16)] = e0
        return carry
    lax.fori_loop(0, C, init_row, 0)

    def zero_buf(r, carry):
        for q in range(W // 16):
            selacc[r, pl.ds(16 * q, 16)] = zeros16
        return carry
    lax.fori_loop(0, EP, zero_buf, 0)

    def zero_acc(b, carry):
        pltpu.sync_copy(selacc, acc_sh.at[pl.ds(r0 + EP * b, EP)])
        return carry
    lax.fori_loop(0, ROWS_PER_TILE // EP, zero_acc, 0)

    pltpu.sync_copy(feat_hbm.at[pl.ds(feat_base + r0, ROWS_PER_TILE)],
                    feat_sh.at[pl.ds(r0, ROWS_PER_TILE)])
    plsc.subcore_barrier()

    def start_gather(jj, b):
        return pltpu.async_copy(
            feat_sh.at[srcbuf.at[jj]], gbufs[b], sems[b])

    def wait_gather(b):
        pltpu.make_async_copy(
            feat_sh.at[srcbuf.at[0]], gbufs[b], sems[b]).wait()

    def superchunk(t, carry):
        g = SUPER_PER_TILE * s + t
        pltpu.sync_copy(src_hbm.at[g], srcbuf)
        pltpu.sync_copy(dst_hbm.at[g], dstbuf)
        pltpu.sync_copy(w_hbm.at[pl.ds(g * SUPER, SUPER)], wbuf)

        start_gather(0, 0)

        def pair(j, cy):
            for b in range(2):
                jj = 2 * j + b
                wait_gather(b)

                @pl.when(jj < K - 1)
                def _():
                    start_gather(jj + 1, 1 - b)

                @pl.when(jj >= 2)
                def _():
                    wait_scatter(b)

                gb = gbufs[b]
                rb = rowbufs[b]
                base16 = jnp.full((16,), jj * C, jnp.int32)

                @functools.partial(plsc.parallel_loop, 0, C, unroll=4)
                def _(i):
                    wv = plsc.load_gather(wbuf, [base16 + i])
                    for q in range(HALF // 16):
                        v = gb[i, pl.ds(16 * q, 16)]
                        rb[i, pl.ds(16 * q, 16)] = v * wv

                pltpu.async_copy(
                    rb, acc_sh.at[dstbuf.at[jj]], ssems[b], add=True)
            return cy
        lax.fori_loop(0, K // 2, pair, 0)
        wait_scatter(0)
        wait_scatter(1)
        return carry

    def wait_scatter(b):
        pltpu.make_async_copy(
            rowbufs[b], acc_sh.at[dstbuf.at[0]], ssems[b]).wait()

    lax.fori_loop(0, SUPER_PER_TILE, superchunk, 0)

    plsc.subcore_barrier()

    tile_slot = c * N_SUB + s

    def ep(b, carry):
        pltpu.sync_copy(acc_sh.at[pl.ds(r0 + EP * b, EP)], selacc)
        pltpu.sync_copy(
            feat_hbm.at[pl.ds(feat_base + r0 + EP * b, EP)], selfeat)

        @functools.partial(plsc.parallel_loop, 0, EP, unroll=5)
        def _(r):
            dvec = selacc[r, pl.ds(HALF, 16)]
            deg = jnp.sum(dvec)
            m = jnp.full((16,), deg, jnp.float32) > 0.0
            for q in range(HALF // 16):
                a = selacc[r, pl.ds(16 * q, 16)]
                f = selfeat[r, pl.ds(16 * q, 16)]
                selfeat[r, pl.ds(16 * q, 16)] = jnp.where(m, a, f)

        pltpu.sync_copy(selfeat, out_hbm.at[tile_slot, pl.ds(EP * b, EP)])
        return carry
    lax.fori_loop(0, ROWS_PER_TILE // EP, ep, 0)


_sc_call = functools.partial(
    pl.kernel,
    out_type=jax.ShapeDtypeStruct((2 * N_SUB, ROWS_PER_TILE, HALF),
                                  jnp.float32),
    mesh=plsc.VectorSubcoreMesh(core_axis_name="c", subcore_axis_name="s"),
    compiler_params=pltpu.CompilerParams(
        needs_layout_passes=False, use_tc_tiling_on_sc=False),
    scratch_types=[
        pltpu.VMEM((K, C), jnp.int32),
        pltpu.VMEM((K, C), jnp.int32),
        pltpu.VMEM((SUPER,), jnp.float32),
        pltpu.VMEM((C, HALF), jnp.float32),
        pltpu.VMEM((C, HALF), jnp.float32),
        pltpu.VMEM((C, W), jnp.float32),
        pltpu.VMEM((C, W), jnp.float32),
        pltpu.VMEM((EP, W), jnp.float32),
        pltpu.VMEM((EP, HALF), jnp.float32),
        pltpu.SemaphoreType.DMA,
        pltpu.SemaphoreType.DMA,
        pltpu.SemaphoreType.DMA,
        pltpu.SemaphoreType.DMA,
        pltpu.VMEM_SHARED((N_NODES, W), jnp.float32),
        pltpu.VMEM_SHARED((N_NODES, HALF), jnp.float32),
    ],
)(_sc_body)


def kernel(features, edge_index, edge_weight):
    ei = edge_index.astype(jnp.int32)
    src3d = ei[0].reshape(E // SUPER, K, C)
    dst3d = ei[1].reshape(E // SUPER, K, C)
    w1d = edge_weight.reshape(E)
    feat_cat = jnp.concatenate([features[:, :HALF], features[:, HALF:]], axis=0)
    out_t = _sc_call(feat_cat, src3d, dst3d, w1d)
    out_cat = out_t.reshape(2, N_NODES, HALF)
    return jnp.concatenate([out_cat[0], out_cat[1]], axis=1)


# speedup vs baseline: 1.0004x; 1.0004x over previous
import functools

import jax
import jax.numpy as jnp
from jax import lax
from jax.experimental import pallas as pl
from jax.experimental.pallas import tpu as pltpu, tpu_sc as plsc

N_NODES = 10000
D = 128
HALF = 64
W = HALF + 16
E = 320000
C = 100
K = 20
SUPER = K * C
N_SUB = 16
SUPER_PER_TILE = E // SUPER // N_SUB
ROWS_PER_TILE = N_NODES // N_SUB
EP = 25


def _sc_body(feat_hbm, src_hbm, dst_hbm, w_hbm, out_hbm,
             srcbuf, dstbuf, wbuf, gbuf0, gbuf1, rowbuf0, rowbuf1,
             selacc, selfeat, sem0, sem1, ssem0, ssem1, acc_sh, feat_sh):
    c = lax.axis_index("c")
    s = lax.axis_index("s")
    r0 = ROWS_PER_TILE * s
    feat_base = c * N_NODES

    zeros16 = jnp.zeros((16,), jnp.float32)
    ones16 = jnp.ones((16,), jnp.float32)
    gbufs = (gbuf0, gbuf1)
    sems = (sem0, sem1)
    rowbufs = (rowbuf0, rowbuf1)
    ssems = (ssem0, ssem1)

    def init_row(i, carry):
        rowbuf0[i, pl.ds(HALF, 16)] = ones16
        rowbuf1[i, pl.ds(HALF, 16)] = ones16
        return carry
    lax.fori_loop(0, C, init_row, 0)

    def zero_buf(r, carry):
        for q in range(W // 16):
            selacc[r, pl.ds(16 * q, 16)] = zeros16
        return carry
    lax.fori_loop(0, EP, zero_buf, 0)

    def zero_acc(b, carry):
        pltpu.sync_copy(selacc, acc_sh.at[pl.ds(r0 + EP * b, EP)])
        return carry
    lax.fori_loop(0, ROWS_PER_TILE // EP, zero_acc, 0)

    pltpu.sync_copy(feat_hbm.at[pl.ds(feat_base + r0, ROWS_PER_TILE)],
                    feat_sh.at[pl.ds(r0, ROWS_PER_TILE)])
    plsc.subcore_barrier()

    def start_gather(jj, b):
        return pltpu.async_copy(
            feat_sh.at[srcbuf.at[jj]], gbufs[b], sems[b])

    def wait_gather(b):
        pltpu.make_async_copy(
            feat_sh.at[srcbuf.at[0]], gbufs[b], sems[b]).wait()

    def superchunk(t, carry):
        g = SUPER_PER_TILE * s + t
        pltpu.sync_copy(src_hbm.at[g], srcbuf)
        pltpu.sync_copy(dst_hbm.at[g], dstbuf)
        pltpu.sync_copy(w_hbm.at[pl.ds(g * SUPER, SUPER)], wbuf)

        start_gather(0, 0)

        def pair(j, cy):
            for b in range(2):
                jj = 2 * j + b
                wait_gather(b)

                @pl.when(jj < K - 1)
                def _():
                    start_gather(jj + 1, 1 - b)

                @pl.when(jj >= 2)
                def _():
                    wait_scatter(b)

                gb = gbufs[b]
                rb = rowbufs[b]
                base16 = jnp.full((16,), jj * C, jnp.int32)

                @functools.partial(plsc.parallel_loop, 0, C, unroll=4)
                def _(i):
                    wv = plsc.load_gather(wbuf, [base16 + i])
                    for q in range(HALF // 16):
                        v = gb[i, pl.ds(16 * q, 16)]
                        rb[i, pl.ds(16 * q, 16)] = v * wv

                pltpu.async_copy(
                    rb, acc_sh.at[dstbuf.at[jj]], ssems[b], add=True)
            return cy
        lax.fori_loop(0, K // 2, pair, 0)
        wait_scatter(0)
        wait_scatter(1)
        return carry

    def wait_scatter(b):
        pltpu.make_async_copy(
            rowbufs[b], acc_sh.at[dstbuf.at[0]], ssems[b]).wait()

    lax.fori_loop(0, SUPER_PER_TILE, superchunk, 0)

    plsc.subcore_barrier()

    tile_slot = c * N_SUB + s

    def ep(b, carry):
        pltpu.sync_copy(acc_sh.at[pl.ds(r0 + EP * b, EP)], selacc)
        pltpu.sync_copy(
            feat_hbm.at[pl.ds(feat_base + r0 + EP * b, EP)], selfeat)

        @functools.partial(plsc.parallel_loop, 0, EP, unroll=5)
        def _(r):
            m = selacc[r, pl.ds(HALF, 16)] > 0.0
            for q in range(HALF // 16):
                a = selacc[r, pl.ds(16 * q, 16)]
                f = selfeat[r, pl.ds(16 * q, 16)]
                selfeat[r, pl.ds(16 * q, 16)] = jnp.where(m, a, f)

        pltpu.sync_copy(selfeat, out_hbm.at[tile_slot, pl.ds(EP * b, EP)])
        return carry
    lax.fori_loop(0, ROWS_PER_TILE // EP, ep, 0)


_sc_call = functools.partial(
    pl.kernel,
    out_type=jax.ShapeDtypeStruct((2 * N_SUB, ROWS_PER_TILE, HALF),
                                  jnp.float32),
    mesh=plsc.VectorSubcoreMesh(core_axis_name="c", subcore_axis_name="s"),
    compiler_params=pltpu.CompilerParams(
        needs_layout_passes=False, use_tc_tiling_on_sc=False),
    scratch_types=[
        pltpu.VMEM((K, C), jnp.int32),
        pltpu.VMEM((K, C), jnp.int32),
        pltpu.VMEM((SUPER,), jnp.float32),
        pltpu.VMEM((C, HALF), jnp.float32),
        pltpu.VMEM((C, HALF), jnp.float32),
        pltpu.VMEM((C, W), jnp.float32),
        pltpu.VMEM((C, W), jnp.float32),
        pltpu.VMEM((EP, W), jnp.float32),
        pltpu.VMEM((EP, HALF), jnp.float32),
        pltpu.SemaphoreType.DMA,
        pltpu.SemaphoreType.DMA,
        pltpu.SemaphoreType.DMA,
        pltpu.SemaphoreType.DMA,
        pltpu.VMEM_SHARED((N_NODES, W), jnp.float32),
        pltpu.VMEM_SHARED((N_NODES, HALF), jnp.float32),
    ],
)(_sc_body)


def kernel(features, edge_index, edge_weight):
    ei = edge_index.astype(jnp.int32)
    src3d = ei[0].reshape(E // SUPER, K, C)
    dst3d = ei[1].reshape(E // SUPER, K, C)
    w1d = edge_weight.reshape(E)
    feat_cat = jnp.concatenate([features[:, :HALF], features[:, HALF:]], axis=0)
    out_t = _sc_call(feat_cat, src3d, dst3d, w1d)
    out_cat = out_t.reshape(2, N_NODES, HALF)
    return jnp.concatenate([out_cat[0], out_cat[1]], axis=1)
